# Initial kernel scaffold; baseline (speedup 1.0000x reference)
import jax
import jax.numpy as jnp
from jax.experimental import pallas as pl


def kernel(x, edge_index, W, b, loop_weight):
    raise NotImplementedError("write your pallas kernel here")


# speedup vs baseline: 2.8571x; 2.8571x over previous
import functools

import jax
import jax.numpy as jnp
from jax import lax
from jax.experimental import pallas as pl
from jax.experimental.pallas import tpu as pltpu
from jax.experimental.pallas import tpu_sc as plsc

N_NODES = 10000
N_EDGES = 160000
F = 256
QF = 64
WID = 72
NQ = 4
EP = 163840
CHUNK = 128
NCH = EP // (16 * CHUNK)
NB = 2
DUMMY = 10100
ROWS_SH = 10240
ZROWS = 640


def _sc_aggregate(xcat, srcoff, dstp, zblk):
    mesh = plsc.VectorSubcoreMesh(core_axis_name="c", subcore_axis_name="s")

    @functools.partial(
        pl.kernel,
        mesh=mesh,
        out_type=jax.ShapeDtypeStruct((NQ * N_NODES, WID), jnp.float32),
        compiler_params=pltpu.CompilerParams(use_tc_tiling_on_sc=False),
        scratch_types=[
            pltpu.VMEM((NCH, CHUNK), jnp.int32),
            pltpu.VMEM((NCH, CHUNK), jnp.int32),
            pltpu.VMEM((CHUNK, WID), jnp.float32),
            pltpu.VMEM((CHUNK, WID), jnp.float32),
            pltpu.VMEM_SHARED((ROWS_SH, WID), jnp.float32),
            pltpu.SemaphoreType.DMA,
            pltpu.SemaphoreType.DMA,
        ],
    )
    def body(xcat_h, srcoff_h, dstp_h, z_h, out_h,
             srcb, dstb, rb0, rb1, agg, sem0, sem1):
        c = lax.axis_index("c")
        s = lax.axis_index("s")
        rbufs = (rb0, rb1)
        sems = (sem0, sem1)

        pltpu.sync_copy(dstp_h.at[pl.ds(s * NCH, NCH)], dstb)

        def gather(ch, b):
            return pltpu.make_async_copy(
                xcat_h.at[srcb.at[ch]], rbufs[b], sems[b])

        def scatter_add(ch, b):
            pltpu.sync_copy(rbufs[b], agg.at[dstb.at[ch]], add=True)

        for p in range(2):
            q = 2 * p + c
            pltpu.sync_copy(z_h, agg.at[pl.ds(s * ZROWS, ZROWS)])
            pltpu.sync_copy(srcoff_h.at[pl.ds((q * 16 + s) * NCH, NCH)], srcb)
            plsc.subcore_barrier()

            for b in range(NB):
                gather(b, b).start()

            def step(i, carry):
                j0 = i * NB
                for b in range(NB):
                    ch = j0 + b
                    gather(ch, b).wait()
                    scatter_add(ch, b)
                    gather(ch + NB, b).start()
                return carry

            lax.fori_loop(0, (NCH - NB) // NB, step, 0)
            for b in range(NB):
                ch = NCH - NB + b
                gather(ch, b).wait()
                scatter_add(ch, b)

            plsc.subcore_barrier()
            @pl.when(s != 15)
            def _():
                pltpu.sync_copy(
                    agg.at[pl.ds(s * ZROWS, ZROWS)],
                    out_h.at[pl.ds(q * N_NODES + s * ZROWS, ZROWS)])

            @pl.when(s == 15)
            def _():
                rem = N_NODES - 15 * ZROWS
                pltpu.sync_copy(
                    agg.at[pl.ds(15 * ZROWS, rem)],
                    out_h.at[pl.ds(q * N_NODES + 15 * ZROWS, rem)])

            plsc.subcore_barrier()

    return body(xcat, srcoff, dstp, zblk)


RB = 1000
QB = N_NODES // RB


def _tc_body(a0, a1, a2, a3, x_ref, w_ref, lw_ref, b_ref, o_ref):
    agg = jnp.concatenate(
        [a0[:, :QF], a1[:, :QF], a2[:, :QF], a3[:, :QF]], axis=1)
    deg = a0[:, QF:QF + 1]
    scale = 1.0 / jnp.maximum(deg, 1.0)
    o_ref[...] = (
        jnp.dot(agg * scale, w_ref[...], preferred_element_type=jnp.float32)
        + jnp.dot(x_ref[...], lw_ref[...], preferred_element_type=jnp.float32)
        + b_ref[...])


def _tc_combine(outcat, x, W, b, loop_weight):
    quarter_spec = lambda q: pl.BlockSpec((RB, WID), lambda i, q=q: (i + q * QB, 0))
    return pl.pallas_call(
        _tc_body,
        grid=(QB,),
        in_specs=[
            quarter_spec(0),
            quarter_spec(1),
            quarter_spec(2),
            quarter_spec(3),
            pl.BlockSpec((RB, F), lambda i: (i, 0)),
            pl.BlockSpec((F, F), lambda i: (0, 0)),
            pl.BlockSpec((F, F), lambda i: (0, 0)),
            pl.BlockSpec((1, F), lambda i: (0, 0)),
        ],
        out_specs=pl.BlockSpec((RB, F), lambda i: (i, 0)),
        out_shape=jax.ShapeDtypeStruct((N_NODES, F), jnp.float32),
    )(outcat, outcat, outcat, outcat, x, W, loop_weight, b.reshape(1, F))


def kernel(x, edge_index, W, b, loop_weight):
    src = edge_index[0].astype(jnp.int32)
    dst = edge_index[1].astype(jnp.int32)
    pad = EP - N_EDGES
    src_p = jnp.concatenate([src, jnp.zeros((pad,), jnp.int32)])
    dst_p = jnp.concatenate([dst, jnp.full((pad,), DUMMY, jnp.int32)])
    srcoff = jnp.concatenate(
        [src_p + q * N_NODES for q in range(NQ)]).reshape(NQ * 16 * NCH, CHUNK)
    dstp = dst_p.reshape(16 * NCH, CHUNK)
    ones = jnp.ones((N_NODES, 1), jnp.float32)
    zpad = jnp.zeros((N_NODES, WID - QF - 1), jnp.float32)
    xcat = jnp.concatenate(
        [jnp.concatenate([x[:, q * QF:(q + 1) * QF], ones, zpad], axis=1)
         for q in range(NQ)], axis=0)
    zblk = jnp.zeros((ZROWS, WID), jnp.float32)
    outcat = _sc_aggregate(xcat, srcoff, dstp, zblk)
    return _tc_combine(outcat, x, W, b, loop_weight)


# speedup vs baseline: 2.8681x; 1.0038x over previous
import functools

import jax
import jax.numpy as jnp
from jax import lax
from jax.experimental import pallas as pl
from jax.experimental.pallas import tpu as pltpu
from jax.experimental.pallas import tpu_sc as plsc

N_NODES = 10000
N_EDGES = 160000
F = 256
QF = 64
WID = 72
NQ = 4
EP = 163840
CHUNK = 128
NCH = EP // (16 * CHUNK)
NB = 4
DUMMY = 10100
ROWS_SH = 10240
ZROWS = 640


def _sc_aggregate(xcat, srcoff, dstp, zblk):
    mesh = plsc.VectorSubcoreMesh(core_axis_name="c", subcore_axis_name="s")

    @functools.partial(
        pl.kernel,
        mesh=mesh,
        out_type=jax.ShapeDtypeStruct((NQ * N_NODES, WID), jnp.float32),
        compiler_params=pltpu.CompilerParams(use_tc_tiling_on_sc=False),
        scratch_types=[
            pltpu.VMEM((NCH, CHUNK), jnp.int32),
            pltpu.VMEM((NCH, CHUNK), jnp.int32),
            pltpu.VMEM((CHUNK, WID), jnp.float32),
            pltpu.VMEM((CHUNK, WID), jnp.float32),
            pltpu.VMEM((CHUNK, WID), jnp.float32),
            pltpu.VMEM((CHUNK, WID), jnp.float32),
            pltpu.VMEM_SHARED((ROWS_SH, WID), jnp.float32),
            pltpu.SemaphoreType.DMA,
            pltpu.SemaphoreType.DMA,
            pltpu.SemaphoreType.DMA,
            pltpu.SemaphoreType.DMA,
            pltpu.SemaphoreType.DMA,
            pltpu.SemaphoreType.DMA,
            pltpu.SemaphoreType.DMA,
            pltpu.SemaphoreType.DMA,
        ],
    )
    def body(xcat_h, srcoff_h, dstp_h, z_h, out_h,
             srcb, dstb, rb0, rb1, rb2, rb3, agg,
             sg0, sg1, sg2, sg3, ss0, ss1, ss2, ss3):
        c = lax.axis_index("c")
        s = lax.axis_index("s")
        rbufs = (rb0, rb1, rb2, rb3)
        gsems = (sg0, sg1, sg2, sg3)
        ssems = (ss0, ss1, ss2, ss3)

        pltpu.sync_copy(dstp_h.at[pl.ds(s * NCH, NCH)], dstb)

        def gfire(ch, b):
            pltpu.async_copy(xcat_h.at[srcb.at[ch]], rbufs[b], gsems[b])

        def gwait(ch, b):
            pltpu.make_async_copy(
                xcat_h.at[srcb.at[ch]], rbufs[b], gsems[b]).wait()

        def sfire(ch, b):
            pltpu.async_copy(rbufs[b], agg.at[dstb.at[ch]], ssems[b],
                             add=True)

        def swait(ch, b):
            pltpu.make_async_copy(
                rbufs[b], agg.at[dstb.at[ch]], ssems[b]).wait()

        for p in range(2):
            q = 2 * p + c
            pltpu.sync_copy(z_h, agg.at[pl.ds(s * ZROWS, ZROWS)])
            pltpu.sync_copy(srcoff_h.at[pl.ds((q * 16 + s) * NCH, NCH)], srcb)
            plsc.subcore_barrier()

            gfire(0, 0)
            gfire(1, 1)

            def step(i, carry):
                j0 = i * NB
                for b in range(NB):
                    ch = j0 + b
                    gwait(ch, b)
                    sfire(ch, b)
                    b2 = (b + 2) % NB

                    @pl.when(ch >= 2)
                    def _():
                        swait(ch - 2, b2)

                    @pl.when(ch + 2 < NCH)
                    def _():
                        gfire(ch + 2, b2)
                return carry

            lax.fori_loop(0, NCH // NB, step, 0)
            swait(NCH - 2, (NCH - 2) % NB)
            swait(NCH - 1, (NCH - 1) % NB)

            plsc.subcore_barrier()
            @pl.when(s != 15)
            def _():
                pltpu.sync_copy(
                    agg.at[pl.ds(s * ZROWS, ZROWS)],
                    out_h.at[pl.ds(q * N_NODES + s * ZROWS, ZROWS)])

            @pl.when(s == 15)
            def _():
                rem = N_NODES - 15 * ZROWS
                pltpu.sync_copy(
                    agg.at[pl.ds(15 * ZROWS, rem)],
                    out_h.at[pl.ds(q * N_NODES + 15 * ZROWS, rem)])

            plsc.subcore_barrier()

    return body(xcat, srcoff, dstp, zblk)


RB = 1000
QB = N_NODES // RB


def _tc_body(a0, a1, a2, a3, x_ref, w_ref, lw_ref, b_ref, o_ref):
    agg = jnp.concatenate(
        [a0[:, :QF], a1[:, :QF], a2[:, :QF], a3[:, :QF]], axis=1)
    deg = a0[:, QF:QF + 1]
    scale = 1.0 / jnp.maximum(deg, 1.0)
    o_ref[...] = (
        jnp.dot(agg * scale, w_ref[...], preferred_element_type=jnp.float32)
        + jnp.dot(x_ref[...], lw_ref[...], preferred_element_type=jnp.float32)
        + b_ref[...])


def _tc_combine(outcat, x, W, b, loop_weight):
    quarter_spec = lambda q: pl.BlockSpec((RB, WID), lambda i, q=q: (i + q * QB, 0))
    return pl.pallas_call(
        _tc_body,
        grid=(QB,),
        in_specs=[
            quarter_spec(0),
            quarter_spec(1),
            quarter_spec(2),
            quarter_spec(3),
            pl.BlockSpec((RB, F), lambda i: (i, 0)),
            pl.BlockSpec((F, F), lambda i: (0, 0)),
            pl.BlockSpec((F, F), lambda i: (0, 0)),
            pl.BlockSpec((1, F), lambda i: (0, 0)),
        ],
        out_specs=pl.BlockSpec((RB, F), lambda i: (i, 0)),
        out_shape=jax.ShapeDtypeStruct((N_NODES, F), jnp.float32),
    )(outcat, outcat, outcat, outcat, x, W, loop_weight, b.reshape(1, F))


def kernel(x, edge_index, W, b, loop_weight):
    src = edge_index[0].astype(jnp.int32)
    dst = edge_index[1].astype(jnp.int32)
    pad = EP - N_EDGES
    src_p = jnp.concatenate([src, jnp.zeros((pad,), jnp.int32)])
    dst_p = jnp.concatenate([dst, jnp.full((pad,), DUMMY, jnp.int32)])
    srcoff = jnp.concatenate(
        [src_p + q * N_NODES for q in range(NQ)]).reshape(NQ * 16 * NCH, CHUNK)
    dstp = dst_p.reshape(16 * NCH, CHUNK)
    ones = jnp.ones((N_NODES, 1), jnp.float32)
    zpad = jnp.zeros((N_NODES, WID - QF - 1), jnp.float32)
    xcat = jnp.concatenate(
        [jnp.concatenate([x[:, q * QF:(q + 1) * QF], ones, zpad], axis=1)
         for q in range(NQ)], axis=0)
    zblk = jnp.zeros((ZROWS, WID), jnp.float32)
    outcat = _sc_aggregate(xcat, srcoff, dstp, zblk)
    return _tc_combine(outcat, x, W, b, loop_weight)


# speedup vs baseline: 4.2993x; 1.4990x over previous
import functools

import jax
import jax.numpy as jnp
from jax import lax
from jax.experimental import pallas as pl
from jax.experimental.pallas import tpu as pltpu
from jax.experimental.pallas import tpu_sc as plsc

N_NODES = 10000
N_EDGES = 160000
F = 256
HF = 128
WID = 144
EP = 163840
CHUNK = 128
NCH = EP // (16 * CHUNK)
NB = 4
DUMMY = 10100
ROWS_SH = 10240
ZROWS = 640


def _sc_aggregate(xcat, srcoff, dstp, zblk):
    mesh = plsc.VectorSubcoreMesh(core_axis_name="c", subcore_axis_name="s")

    @functools.partial(
        pl.kernel,
        mesh=mesh,
        out_type=jax.ShapeDtypeStruct((2 * N_NODES, WID), jnp.bfloat16),
        compiler_params=pltpu.CompilerParams(use_tc_tiling_on_sc=False),
        scratch_types=[
            pltpu.VMEM((NCH, CHUNK), jnp.int32),
            pltpu.VMEM((NCH, CHUNK), jnp.int32),
            pltpu.VMEM((CHUNK, WID), jnp.bfloat16),
            pltpu.VMEM((CHUNK, WID), jnp.bfloat16),
            pltpu.VMEM((CHUNK, WID), jnp.bfloat16),
            pltpu.VMEM((CHUNK, WID), jnp.bfloat16),
            pltpu.VMEM_SHARED((ROWS_SH, WID), jnp.bfloat16),
            pltpu.SemaphoreType.DMA,
            pltpu.SemaphoreType.DMA,
            pltpu.SemaphoreType.DMA,
            pltpu.SemaphoreType.DMA,
            pltpu.SemaphoreType.DMA,
            pltpu.SemaphoreType.DMA,
            pltpu.SemaphoreType.DMA,
            pltpu.SemaphoreType.DMA,
        ],
    )
    def body(xcat_h, srcoff_h, dstp_h, z_h, out_h,
             srcb, dstb, rb0, rb1, rb2, rb3, agg,
             sg0, sg1, sg2, sg3, ss0, ss1, ss2, ss3):
        c = lax.axis_index("c")
        s = lax.axis_index("s")
        rbufs = (rb0, rb1, rb2, rb3)
        gsems = (sg0, sg1, sg2, sg3)
        ssems = (ss0, ss1, ss2, ss3)

        pltpu.sync_copy(z_h, agg.at[pl.ds(s * ZROWS, ZROWS)])
        pltpu.sync_copy(srcoff_h.at[pl.ds((c * 16 + s) * NCH, NCH)], srcb)
        pltpu.sync_copy(dstp_h.at[pl.ds(s * NCH, NCH)], dstb)
        plsc.subcore_barrier()

        def gfire(ch, b):
            pltpu.async_copy(xcat_h.at[srcb.at[ch]], rbufs[b], gsems[b])

        def gwait(ch, b):
            pltpu.make_async_copy(
                xcat_h.at[srcb.at[ch]], rbufs[b], gsems[b]).wait()

        def sfire(ch, b):
            pltpu.async_copy(rbufs[b], agg.at[dstb.at[ch]], ssems[b],
                             add=True)

        def swait(ch, b):
            pltpu.make_async_copy(
                rbufs[b], agg.at[dstb.at[ch]], ssems[b]).wait()

        gfire(0, 0)
        gfire(1, 1)

        def step(i, carry):
            j0 = i * NB
            for b in range(NB):
                ch = j0 + b
                gwait(ch, b)
                sfire(ch, b)
                b2 = (b + 2) % NB

                @pl.when(ch >= 2)
                def _():
                    swait(ch - 2, b2)

                @pl.when(ch + 2 < NCH)
                def _():
                    gfire(ch + 2, b2)
            return carry

        lax.fori_loop(0, NCH // NB, step, 0)
        swait(NCH - 2, (NCH - 2) % NB)
        swait(NCH - 1, (NCH - 1) % NB)

        plsc.subcore_barrier()
        @pl.when(s != 15)
        def _():
            pltpu.sync_copy(
                agg.at[pl.ds(s * ZROWS, ZROWS)],
                out_h.at[pl.ds(c * N_NODES + s * ZROWS, ZROWS)])

        @pl.when(s == 15)
        def _():
            rem = N_NODES - 15 * ZROWS
            pltpu.sync_copy(
                agg.at[pl.ds(15 * ZROWS, rem)],
                out_h.at[pl.ds(c * N_NODES + 15 * ZROWS, rem)])

    return body(xcat, srcoff, dstp, zblk)


RB = 1000
HB = N_NODES // RB


def _tc_body(a0, a1, x_ref, w_ref, lw_ref, b_ref, o_ref):
    agg = jnp.concatenate(
        [a0[:, :HF], a1[:, :HF]], axis=1).astype(jnp.float32)
    deg = a0[:, HF:HF + 1].astype(jnp.float32)
    scale = 1.0 / jnp.maximum(deg, 1.0)
    o_ref[...] = (
        jnp.dot(agg * scale, w_ref[...], preferred_element_type=jnp.float32)
        + jnp.dot(x_ref[...], lw_ref[...], preferred_element_type=jnp.float32)
        + b_ref[...])


def _tc_combine(outcat, x, W, b, loop_weight):
    half_spec = lambda h: pl.BlockSpec((RB, WID), lambda i, h=h: (i + h * HB, 0))
    return pl.pallas_call(
        _tc_body,
        grid=(HB,),
        in_specs=[
            half_spec(0),
            half_spec(1),
            pl.BlockSpec((RB, F), lambda i: (i, 0)),
            pl.BlockSpec((F, F), lambda i: (0, 0)),
            pl.BlockSpec((F, F), lambda i: (0, 0)),
            pl.BlockSpec((1, F), lambda i: (0, 0)),
        ],
        out_specs=pl.BlockSpec((RB, F), lambda i: (i, 0)),
        out_shape=jax.ShapeDtypeStruct((N_NODES, F), jnp.float32),
    )(outcat, outcat, x, W, loop_weight, b.reshape(1, F))


def kernel(x, edge_index, W, b, loop_weight):
    src = edge_index[0].astype(jnp.int32)
    dst = edge_index[1].astype(jnp.int32)
    pad = EP - N_EDGES
    src_p = jnp.concatenate([src, jnp.zeros((pad,), jnp.int32)])
    dst_p = jnp.concatenate([dst, jnp.full((pad,), DUMMY, jnp.int32)])
    srcoff = jnp.concatenate(
        [src_p, src_p + N_NODES]).reshape(2 * 16 * NCH, CHUNK)
    dstp = dst_p.reshape(16 * NCH, CHUNK)
    ones = jnp.ones((N_NODES, 1), jnp.float32)
    zpad = jnp.zeros((N_NODES, WID - HF - 1), jnp.float32)
    xcat = jnp.concatenate(
        [jnp.concatenate([x[:, h * HF:(h + 1) * HF], ones, zpad], axis=1)
         for h in range(2)], axis=0).astype(jnp.bfloat16)
    zblk = jnp.zeros((ZROWS, WID), jnp.bfloat16)
    outcat = _sc_aggregate(xcat, srcoff, dstp, zblk)
    return _tc_combine(outcat, x, W, b, loop_weight)


# speedup vs baseline: 4.4626x; 1.0380x over previous
import functools

import jax
import jax.numpy as jnp
from jax import lax
from jax.experimental import pallas as pl
from jax.experimental.pallas import tpu as pltpu
from jax.experimental.pallas import tpu_sc as plsc

N_NODES = 10000
N_EDGES = 160000
F = 256
HF = 128
WID = 144
EP = 163840
CHUNK = 128
NCH = EP // (16 * CHUNK)
NB = 4
DUMMY = 10100
ROWS_SH = 10240
ZROWS = 640


def _sc_aggregate(xcat, srcoff, dstp, zblk):
    mesh = plsc.VectorSubcoreMesh(core_axis_name="c", subcore_axis_name="s")

    @functools.partial(
        pl.kernel,
        mesh=mesh,
        out_type=jax.ShapeDtypeStruct((2 * N_NODES, WID), jnp.bfloat16),
        compiler_params=pltpu.CompilerParams(use_tc_tiling_on_sc=False),
        scratch_types=[
            pltpu.VMEM((NCH, CHUNK), jnp.int32),
            pltpu.VMEM((NCH, CHUNK), jnp.int32),
            pltpu.VMEM((CHUNK, WID), jnp.bfloat16),
            pltpu.VMEM((CHUNK, WID), jnp.bfloat16),
            pltpu.VMEM((CHUNK, WID), jnp.bfloat16),
            pltpu.VMEM((CHUNK, WID), jnp.bfloat16),
            pltpu.VMEM_SHARED((ROWS_SH, WID), jnp.bfloat16),
            pltpu.SemaphoreType.DMA,
            pltpu.SemaphoreType.DMA,
            pltpu.SemaphoreType.DMA,
            pltpu.SemaphoreType.DMA,
            pltpu.SemaphoreType.DMA,
            pltpu.SemaphoreType.DMA,
            pltpu.SemaphoreType.DMA,
            pltpu.SemaphoreType.DMA,
        ],
    )
    def body(xcat_h, srcoff_h, dstp_h, z_h, out_h,
             srcb, dstb, rb0, rb1, rb2, rb3, agg,
             sg0, sg1, sg2, sg3, ss0, ss1, ss2, ss3):
        c = lax.axis_index("c")
        s = lax.axis_index("s")
        rbufs = (rb0, rb1, rb2, rb3)
        gsems = (sg0, sg1, sg2, sg3)
        ssems = (ss0, ss1, ss2, ss3)

        pltpu.sync_copy(z_h, agg.at[pl.ds(s * ZROWS, ZROWS)])
        pltpu.sync_copy(srcoff_h.at[pl.ds((c * 16 + s) * NCH, NCH)], srcb)
        pltpu.sync_copy(dstp_h.at[pl.ds(s * NCH, NCH)], dstb)
        plsc.subcore_barrier()

        def gfire(ch, b):
            pltpu.async_copy(xcat_h.at[srcb.at[ch]], rbufs[b], gsems[b])

        def gwait(ch, b):
            pltpu.make_async_copy(
                xcat_h.at[srcb.at[ch]], rbufs[b], gsems[b]).wait()

        def sfire(ch, b):
            pltpu.async_copy(rbufs[b], agg.at[dstb.at[ch]], ssems[b],
                             add=True)

        def swait(ch, b):
            pltpu.make_async_copy(
                rbufs[b], agg.at[dstb.at[ch]], ssems[b]).wait()

        gfire(0, 0)
        gfire(1, 1)

        def step(i, carry):
            j0 = i * NB
            for b in range(NB):
                ch = j0 + b
                gwait(ch, b)
                sfire(ch, b)
                b2 = (b + 2) % NB

                @pl.when(ch >= 2)
                def _():
                    swait(ch - 2, b2)

                @pl.when(ch + 2 < NCH)
                def _():
                    gfire(ch + 2, b2)
            return carry

        lax.fori_loop(0, NCH // NB, step, 0)
        swait(NCH - 2, (NCH - 2) % NB)
        swait(NCH - 1, (NCH - 1) % NB)

        plsc.subcore_barrier()
        @pl.when(s != 15)
        def _():
            pltpu.sync_copy(
                agg.at[pl.ds(s * ZROWS, ZROWS)],
                out_h.at[pl.ds(c * N_NODES + s * ZROWS, ZROWS)])

        @pl.when(s == 15)
        def _():
            rem = N_NODES - 15 * ZROWS
            pltpu.sync_copy(
                agg.at[pl.ds(15 * ZROWS, rem)],
                out_h.at[pl.ds(c * N_NODES + 15 * ZROWS, rem)])

    return body(xcat, srcoff, dstp, zblk)


RB = 1000
HB = N_NODES // RB


def _build_body(x_ref, o_ref):
    h = pl.program_id(0)
    i = pl.program_id(1)
    del i
    xb = x_ref[...].astype(jnp.bfloat16)
    ones = jnp.ones((RB, 1), jnp.bfloat16)
    zpad = jnp.zeros((RB, WID - HF - 1), jnp.bfloat16)
    del h
    o_ref[...] = jnp.concatenate([xb, ones, zpad], axis=1)


def _tc_build_table(x):
    return pl.pallas_call(
        _build_body,
        grid=(2, HB),
        in_specs=[pl.BlockSpec((RB, HF), lambda h, i: (i, h))],
        out_specs=pl.BlockSpec((RB, WID), lambda h, i: (h * HB + i, 0)),
        out_shape=jax.ShapeDtypeStruct((2 * N_NODES, WID), jnp.bfloat16),
    )(x)


def _selfloop_body(x_ref, lw_ref, b_ref, o_ref):
    o_ref[...] = jnp.dot(x_ref[...], lw_ref[...],
                         preferred_element_type=jnp.float32) + b_ref[...]


def _tc_selfloop(x, loop_weight, b):
    return pl.pallas_call(
        _selfloop_body,
        grid=(HB,),
        in_specs=[
            pl.BlockSpec((RB, F), lambda i: (i, 0)),
            pl.BlockSpec((F, F), lambda i: (0, 0)),
            pl.BlockSpec((1, F), lambda i: (0, 0)),
        ],
        out_specs=pl.BlockSpec((RB, F), lambda i: (i, 0)),
        out_shape=jax.ShapeDtypeStruct((N_NODES, F), jnp.float32),
    )(x, loop_weight, b.reshape(1, F))


def _tc_body(a0, a1, sl_ref, w_ref, o_ref):
    agg = jnp.concatenate(
        [a0[:, :HF], a1[:, :HF]], axis=1).astype(jnp.float32)
    deg = a0[:, HF:HF + 1].astype(jnp.float32)
    scale = 1.0 / jnp.maximum(deg, 1.0)
    o_ref[...] = jnp.dot(agg * scale, w_ref[...],
                         preferred_element_type=jnp.float32) + sl_ref[...]


def _tc_combine(outcat, selfloop, W):
    half_spec = lambda h: pl.BlockSpec((RB, WID), lambda i, h=h: (i + h * HB, 0))
    return pl.pallas_call(
        _tc_body,
        grid=(HB,),
        in_specs=[
            half_spec(0),
            half_spec(1),
            pl.BlockSpec((RB, F), lambda i: (i, 0)),
            pl.BlockSpec((F, F), lambda i: (0, 0)),
        ],
        out_specs=pl.BlockSpec((RB, F), lambda i: (i, 0)),
        out_shape=jax.ShapeDtypeStruct((N_NODES, F), jnp.float32),
    )(outcat, outcat, selfloop, W)


def kernel(x, edge_index, W, b, loop_weight):
    src = edge_index[0].astype(jnp.int32)
    dst = edge_index[1].astype(jnp.int32)
    pad = EP - N_EDGES
    src_p = jnp.concatenate([src, jnp.zeros((pad,), jnp.int32)])
    dst_p = jnp.concatenate([dst, jnp.full((pad,), DUMMY, jnp.int32)])
    srcoff = jnp.concatenate(
        [src_p, src_p + N_NODES]).reshape(2 * 16 * NCH, CHUNK)
    dstp = dst_p.reshape(16 * NCH, CHUNK)
    xcat = _tc_build_table(x)
    zblk = jnp.zeros((ZROWS, WID), jnp.bfloat16)
    outcat = _sc_aggregate(xcat, srcoff, dstp, zblk)
    selfloop = _tc_selfloop(x, loop_weight, b)
    return _tc_combine(outcat, selfloop, W)


# speedup vs baseline: 4.5002x; 1.0084x over previous
import functools

import jax
import jax.numpy as jnp
from jax import lax
from jax.experimental import pallas as pl
from jax.experimental.pallas import tpu as pltpu
from jax.experimental.pallas import tpu_sc as plsc

N_NODES = 10000
N_EDGES = 160000
F = 256
HF = 128
WID = 144
EP = 163840
CHUNK = 128
NCH = EP // (16 * CHUNK)
NB = 4
DUMMY = 10100
ROWS_SH = 10240
ZROWS = 640


def _sc_aggregate(xcat, srcoff, dstp, zblk):
    mesh = plsc.VectorSubcoreMesh(core_axis_name="c", subcore_axis_name="s")

    @functools.partial(
        pl.kernel,
        mesh=mesh,
        out_type=jax.ShapeDtypeStruct((2 * N_NODES, WID), jnp.bfloat16),
        compiler_params=pltpu.CompilerParams(use_tc_tiling_on_sc=False),
        scratch_types=[
            pltpu.VMEM((NCH, CHUNK), jnp.int32),
            pltpu.VMEM((NCH, CHUNK), jnp.int32),
            pltpu.VMEM((CHUNK, WID), jnp.bfloat16),
            pltpu.VMEM((CHUNK, WID), jnp.bfloat16),
            pltpu.VMEM((CHUNK, WID), jnp.bfloat16),
            pltpu.VMEM((CHUNK, WID), jnp.bfloat16),
            pltpu.VMEM_SHARED((ROWS_SH, WID), jnp.bfloat16),
            pltpu.SemaphoreType.DMA,
            pltpu.SemaphoreType.DMA,
            pltpu.SemaphoreType.DMA,
            pltpu.SemaphoreType.DMA,
            pltpu.SemaphoreType.DMA,
            pltpu.SemaphoreType.DMA,
            pltpu.SemaphoreType.DMA,
            pltpu.SemaphoreType.DMA,
        ],
    )
    def body(xcat_h, srcoff_h, dstp_h, z_h, out_h,
             srcb, dstb, rb0, rb1, rb2, rb3, agg,
             sg0, sg1, sg2, sg3, ss0, ss1, ss2, ss3):
        c = lax.axis_index("c")
        s = lax.axis_index("s")
        rbufs = (rb0, rb1, rb2, rb3)
        gsems = (sg0, sg1, sg2, sg3)
        ssems = (ss0, ss1, ss2, ss3)

        pltpu.sync_copy(z_h, agg.at[pl.ds(s * ZROWS, ZROWS)])
        pltpu.sync_copy(srcoff_h.at[pl.ds((c * 16 + s) * NCH, NCH)], srcb)
        pltpu.sync_copy(dstp_h.at[pl.ds(s * NCH, NCH)], dstb)
        plsc.subcore_barrier()

        def gfire(ch, b):
            pltpu.async_copy(xcat_h.at[srcb.at[ch]], rbufs[b], gsems[b])

        def gwait(ch, b):
            pltpu.make_async_copy(
                xcat_h.at[srcb.at[ch]], rbufs[b], gsems[b]).wait()

        def sfire(ch, b):
            pltpu.async_copy(rbufs[b], agg.at[dstb.at[ch]], ssems[b],
                             add=True)

        def swait(ch, b):
            pltpu.make_async_copy(
                rbufs[b], agg.at[dstb.at[ch]], ssems[b]).wait()

        gfire(0, 0)
        gfire(1, 1)

        def step(i, carry):
            j0 = i * NB
            for b in range(NB):
                ch = j0 + b
                gwait(ch, b)
                sfire(ch, b)
                b2 = (b + 2) % NB

                @pl.when(ch >= 2)
                def _():
                    swait(ch - 2, b2)

                @pl.when(ch + 2 < NCH)
                def _():
                    gfire(ch + 2, b2)
            return carry

        lax.fori_loop(0, NCH // NB, step, 0)
        swait(NCH - 2, (NCH - 2) % NB)
        swait(NCH - 1, (NCH - 1) % NB)

        plsc.subcore_barrier()
        @pl.when(s != 15)
        def _():
            pltpu.sync_copy(
                agg.at[pl.ds(s * ZROWS, ZROWS)],
                out_h.at[pl.ds(c * N_NODES + s * ZROWS, ZROWS)])

        @pl.when(s == 15)
        def _():
            rem = N_NODES - 15 * ZROWS
            pltpu.sync_copy(
                agg.at[pl.ds(15 * ZROWS, rem)],
                out_h.at[pl.ds(c * N_NODES + 15 * ZROWS, rem)])

    return body(xcat, srcoff, dstp, zblk)


RB = 1000
HB = N_NODES // RB


def _build_body(x_ref, o_ref):
    h = pl.program_id(0)
    i = pl.program_id(1)
    del i
    xb = x_ref[...].astype(jnp.bfloat16)
    ones = jnp.ones((RB, 1), jnp.bfloat16)
    zpad = jnp.zeros((RB, WID - HF - 1), jnp.bfloat16)
    del h
    o_ref[...] = jnp.concatenate([xb, ones, zpad], axis=1)


def _tc_build_table(x):
    return pl.pallas_call(
        _build_body,
        grid=(2, HB),
        in_specs=[pl.BlockSpec((RB, HF), lambda h, i: (i, h))],
        out_specs=pl.BlockSpec((RB, WID), lambda h, i: (h * HB + i, 0)),
        out_shape=jax.ShapeDtypeStruct((2 * N_NODES, WID), jnp.bfloat16),
    )(x)


def _tc_body(a0, a1, x_ref, w_ref, lw_ref, b_ref, o_ref):
    agg = jnp.concatenate(
        [a0[:, :HF], a1[:, :HF]], axis=1).astype(jnp.float32)
    deg = a0[:, HF:HF + 1].astype(jnp.float32)
    scale = 1.0 / jnp.maximum(deg, 1.0)
    o_ref[...] = (
        jnp.dot(agg * scale, w_ref[...], preferred_element_type=jnp.float32)
        + jnp.dot(x_ref[...], lw_ref[...], preferred_element_type=jnp.float32)
        + b_ref[...])


def _tc_combine(outcat, x, W, b, loop_weight):
    half_spec = lambda h: pl.BlockSpec((RB, WID), lambda i, h=h: (i + h * HB, 0))
    return pl.pallas_call(
        _tc_body,
        grid=(HB,),
        in_specs=[
            half_spec(0),
            half_spec(1),
            pl.BlockSpec((RB, F), lambda i: (i, 0)),
            pl.BlockSpec((F, F), lambda i: (0, 0)),
            pl.BlockSpec((F, F), lambda i: (0, 0)),
            pl.BlockSpec((1, F), lambda i: (0, 0)),
        ],
        out_specs=pl.BlockSpec((RB, F), lambda i: (i, 0)),
        out_shape=jax.ShapeDtypeStruct((N_NODES, F), jnp.float32),
    )(outcat, outcat, x, W, loop_weight, b.reshape(1, F))


def kernel(x, edge_index, W, b, loop_weight):
    src = edge_index[0].astype(jnp.int32)
    dst = edge_index[1].astype(jnp.int32)
    pad = EP - N_EDGES
    src_p = jnp.concatenate([src, jnp.zeros((pad,), jnp.int32)])
    dst_p = jnp.concatenate([dst, jnp.full((pad,), DUMMY, jnp.int32)])
    srcoff = jnp.concatenate(
        [src_p, src_p + N_NODES]).reshape(2 * 16 * NCH, CHUNK)
    dstp = dst_p.reshape(16 * NCH, CHUNK)
    xcat = _tc_build_table(x)
    zblk = jnp.zeros((ZROWS, WID), jnp.bfloat16)
    outcat = _sc_aggregate(xcat, srcoff, dstp, zblk)
    return _tc_combine(outcat, x, W, b, loop_weight)
